# baseline (speedup 1.0000x reference)
import functools

import jax
import jax.numpy as jnp
from jax import lax
from jax.experimental import pallas as pl
from jax.experimental.pallas import tpu as pltpu
from jax.experimental.pallas import tpu_sc as plsc

Q_LEN = 1
L = 32
BSZ = 64
NHID = 256
N = 32
DK = L * NHID
TOPK = 8
LANES = 16
NC = 2
NS = 16
NW = NC * NS
B_PER_W = BSZ // NW
SCALE = 1.0 / float(DK) ** 0.5

QTR = DK // 4
SG = 4
NGROUPS = N // SG


def _round_bf16_pair(x, y):
    pu = plsc.bitcast(plsc.pack(x, y, format=plsc.PackFormat.INTERLEAVED),
                      jnp.uint32)
    xr = plsc.bitcast(pu << 16, jnp.float32)
    yr = plsc.bitcast(pu & jnp.uint32(0xFFFF0000), jnp.float32)
    return xr, yr


def _sc_body(q_hbm, keys4_hbm, attn_hbm, topk_hbm, qv, kb0, kb1, idxr, sv, tv,
             sem_q, sem0, sem1):
    wid = lax.axis_index("s") * NC + lax.axis_index("c")
    idx0 = lax.iota(jnp.int32, LANES)
    idx1 = idx0 + LANES
    kbufs = (kb0, kb1)
    sems = (sem0, sem1)

    for rep in range(B_PER_W):
        b = wid * B_PER_W + rep

        qcps = [pltpu.async_copy(q_hbm.at[0, l, b, :],
                                 qv.at[pl.ds(l * NHID, NHID)], sem_q)
                for l in range(L)]

        def fire(sg):
            p = sg % 2
            idxr[p, :] = (sg * (SG * BSZ * 4) + (idx0 & 3) * (BSZ * 4)
                          + b * 4 + (idx0 >> 2))
            return pltpu.async_copy(keys4_hbm.at[idxr.at[p]], kbufs[p],
                                    sems[p])

        cps = {0: fire(0)}

        for cp in qcps:
            cp.wait()

        @plsc.parallel_loop(0, DK // (2 * LANES), unroll=4)
        def q_round_body(i):
            q0, q1 = _round_bf16_pair(qv[pl.ds(i * 2 * LANES, LANES)],
                                      qv[pl.ds(i * 2 * LANES + LANES, LANES)])
            qv[pl.ds(i * 2 * LANES, LANES)] = q0
            qv[pl.ds(i * 2 * LANES + LANES, LANES)] = q1

        s0 = jnp.zeros((LANES,), jnp.float32)
        s1 = jnp.zeros((LANES,), jnp.float32)
        for sg in range(NGROUPS):
            if sg + 1 < NGROUPS:
                cps[sg + 1] = fire(sg + 1)
            cps.pop(sg).wait()
            buf = kbufs[sg % 2]
            accs = (jnp.zeros((LANES,), jnp.float32),) * SG

            def dot_body(i, accs):
                qtr = i >> 6
                ii = i & 63
                qo = qtr * QTR + ii * 2 * LANES
                col = ii * 2 * LANES
                base = qtr * SG
                q0 = qv[pl.ds(qo, LANES)]
                q1 = qv[pl.ds(qo + LANES, LANES)]
                out = []
                for s in range(SG):
                    k0, k1 = _round_bf16_pair(
                        buf[base + s, pl.ds(col, LANES)],
                        buf[base + s, pl.ds(col + LANES, LANES)])
                    out.append(accs[s] + k0 * q0 + k1 * q1)
                return tuple(out)

            accs = plsc.parallel_loop(0, 4 * (QTR // (2 * LANES)), unroll=4,
                                      carry=accs)(dot_body)
            for s in range(SG):
                n = sg * SG + s
                score = jnp.sum(accs[s]) * SCALE
                if n < LANES:
                    s0 = jnp.where(idx0 == n, score, s0)
                else:
                    s1 = jnp.where(idx0 == (n - LANES), score, s1)

        m = jnp.maximum(jnp.max(s0), jnp.max(s1))
        e0 = jnp.exp(s0 - m)
        e1 = jnp.exp(s1 - m)
        denom = jnp.sum(e0) + jnp.sum(e1)
        a0 = e0 / denom
        a1 = e1 / denom

        sv[pl.ds(0, LANES)] = a0
        sv[pl.ds(LANES, LANES)] = a1
        pltpu.sync_copy(sv, attn_hbm.at[pl.ds(b * N, N)])

        r0 = jnp.zeros((LANES,), jnp.int32)
        r1 = jnp.zeros((LANES,), jnp.int32)
        for mi in range(N):
            am_s = a0[mi] if mi < LANES else a1[mi - LANES]
            am = jnp.broadcast_to(am_s, (LANES,))
            r0 = r0 + (am > a0).astype(jnp.int32)
            r1 = r1 + (am > a1).astype(jnp.int32)
            r0 = r0 + ((am == a0) & (idx0 > mi)).astype(jnp.int32)
            r1 = r1 + ((am == a1) & (idx1 > mi)).astype(jnp.int32)

        plsc.store_scatter(tv, [r0], idx0, mask=r0 < TOPK)
        plsc.store_scatter(tv, [r1], idx1, mask=r1 < TOPK)
        pltpu.sync_copy(tv.at[pl.ds(0, TOPK)], topk_hbm.at[pl.ds(b * TOPK, TOPK)])


@functools.partial(
    pl.kernel,
    mesh=plsc.VectorSubcoreMesh(core_axis_name="c", subcore_axis_name="s"),
    out_type=[
        jax.ShapeDtypeStruct((BSZ * N,), jnp.float32),
        jax.ShapeDtypeStruct((BSZ * TOPK,), jnp.int32),
    ],
    scratch_types=[
        pltpu.VMEM((DK,), jnp.float32),
        pltpu.VMEM((4 * SG, QTR), jnp.float32),
        pltpu.VMEM((4 * SG, QTR), jnp.float32),
        pltpu.VMEM((2, LANES), jnp.int32),
        pltpu.VMEM((N,), jnp.float32),
        pltpu.VMEM((LANES,), jnp.int32),
        pltpu.SemaphoreType.DMA,
        pltpu.SemaphoreType.DMA,
        pltpu.SemaphoreType.DMA,
    ],
    compiler_params=pltpu.CompilerParams(needs_layout_passes=False),
)
def _sc_cache_attn(q_hbm, keys4_hbm, attn_hbm, topk_hbm, qv, kb0, kb1, idxr,
                   sv, tv, sem_q, sem0, sem1):
    _sc_body(q_hbm, keys4_hbm, attn_hbm, topk_hbm, qv, kb0, kb1, idxr, sv, tv,
             sem_q, sem0, sem1)


def kernel(query, keys, values):
    del values
    keys4 = keys.reshape(N * BSZ * 4, QTR)
    attn_flat, topk_flat = _sc_cache_attn(query, keys4)
    attention = attn_flat.reshape(BSZ, 1, N)
    topk_indices = topk_flat.reshape(BSZ, TOPK).T
    return attention, topk_indices


# speedup vs baseline: 2.0943x; 2.0943x over previous
import functools

import jax
import jax.numpy as jnp
from jax import lax
from jax.experimental import pallas as pl
from jax.experimental.pallas import tpu as pltpu
from jax.experimental.pallas import tpu_sc as plsc

Q_LEN = 1
L = 32
BSZ = 64
NHID = 256
N = 32
DK = L * NHID
TOPK = 8
LANES = 16
NC = 2
NS = 16
NW = NC * NS
B_PER_W = BSZ // NW
SCALE = 1.0 / float(DK) ** 0.5


def _scores_body(keys_ref, q_ref, out_ref, scores_ref):
    n = pl.program_id(0)
    kr = keys_ref[0].astype(jnp.bfloat16).astype(jnp.float32)
    qr = q_ref[...].astype(jnp.float32)
    partial = jnp.sum(kr * qr, axis=1) * SCALE
    lane = jax.lax.broadcasted_iota(jnp.int32, (BSZ, N), 1)
    scores_ref[...] = jnp.where(lane == n, partial[:, None], scores_ref[...])

    @pl.when(n == N - 1)
    def _():
        s = scores_ref[...]
        m = jnp.max(s, axis=1, keepdims=True)
        e = jnp.exp(s - m)
        out_ref[...] = (e / jnp.sum(e, axis=1, keepdims=True))[:, None, :]


_scores_call = pl.pallas_call(
    _scores_body,
    grid=(N,),
    in_specs=[
        pl.BlockSpec((1, BSZ, DK), lambda n: (n, 0, 0)),
        pl.BlockSpec((BSZ, DK), lambda n: (0, 0)),
    ],
    out_specs=pl.BlockSpec((BSZ, 1, N), lambda n: (0, 0, 0)),
    out_shape=jax.ShapeDtypeStruct((BSZ, 1, N), jnp.float32),
    scratch_shapes=[pltpu.VMEM((BSZ, N), jnp.float32)],
)


def _topk_body(attn_hbm, topk_hbm, sv, tv, sem):
    wid = lax.axis_index("s") * NC + lax.axis_index("c")
    idx0 = lax.iota(jnp.int32, LANES)
    idx1 = idx0 + LANES

    for rep in range(B_PER_W):
        b = wid * B_PER_W + rep
        pltpu.sync_copy(attn_hbm.at[pl.ds(b * N, N)], sv)
        a0 = sv[pl.ds(0, LANES)]
        a1 = sv[pl.ds(LANES, LANES)]

        r0 = jnp.zeros((LANES,), jnp.int32)
        r1 = jnp.zeros((LANES,), jnp.int32)
        for mi in range(N):
            am_s = a0[mi] if mi < LANES else a1[mi - LANES]
            am = jnp.broadcast_to(am_s, (LANES,))
            r0 = r0 + (am > a0).astype(jnp.int32)
            r1 = r1 + (am > a1).astype(jnp.int32)
            r0 = r0 + ((am == a0) & (idx0 > mi)).astype(jnp.int32)
            r1 = r1 + ((am == a1) & (idx1 > mi)).astype(jnp.int32)

        plsc.store_scatter(tv, [r0], idx0, mask=r0 < TOPK)
        plsc.store_scatter(tv, [r1], idx1, mask=r1 < TOPK)
        pltpu.sync_copy(tv.at[pl.ds(0, TOPK)], topk_hbm.at[pl.ds(b * TOPK, TOPK)])


@functools.partial(
    pl.kernel,
    mesh=plsc.VectorSubcoreMesh(core_axis_name="c", subcore_axis_name="s"),
    out_type=jax.ShapeDtypeStruct((BSZ * TOPK,), jnp.int32),
    scratch_types=[
        pltpu.VMEM((N,), jnp.float32),
        pltpu.VMEM((LANES,), jnp.int32),
        pltpu.SemaphoreType.DMA,
    ],
    compiler_params=pltpu.CompilerParams(needs_layout_passes=False),
)
def _sc_topk(attn_hbm, topk_hbm, sv, tv, sem):
    _topk_body(attn_hbm, topk_hbm, sv, tv, sem)


def kernel(query, keys, values):
    del values
    qf = jnp.transpose(query, (0, 2, 1, 3)).reshape(BSZ, DK)
    qb16 = qf.astype(jnp.bfloat16)
    attention = _scores_call(keys, qb16)
    topk_flat = _sc_topk(attention.reshape(BSZ * N))
    topk_indices = topk_flat.reshape(BSZ, TOPK).T
    return attention, topk_indices


# speedup vs baseline: 2.3523x; 1.1232x over previous
import functools

import jax
import jax.numpy as jnp
from jax import lax
from jax.experimental import pallas as pl
from jax.experimental.pallas import tpu as pltpu
from jax.experimental.pallas import tpu_sc as plsc

Q_LEN = 1
L = 32
BSZ = 64
NHID = 256
N = 32
DK = L * NHID
TOPK = 8
LANES = 16
NC = 2
NS = 16
NW = NC * NS
SCALE = 1.0 / float(DK) ** 0.5

B_TC = 32
B_SC = BSZ - B_TC

SG = 8
DH = 4096
NH = DK // DH
NGROUPS = N // SG


def _round_bf16_pair(x, y):
    pu = plsc.bitcast(plsc.pack(x, y, format=plsc.PackFormat.INTERLEAVED),
                      jnp.uint32)
    xr = plsc.bitcast(pu << 16, jnp.float32)
    yr = plsc.bitcast(pu & jnp.uint32(0xFFFF0000), jnp.float32)
    return xr, yr


def _sc_body(q_hbm, keys_hbm, attn_hbm, topk_hbm, qv, kb0, kb1, sv, tv,
             sem_q, sem0, sem1):
    wid = lax.axis_index("s") * NC + lax.axis_index("c")
    idx0 = lax.iota(jnp.int32, LANES)
    idx1 = idx0 + LANES
    kbufs = (kb0, kb1)
    sems = (sem0, sem1)

    b = B_TC + wid
    qcps = [pltpu.async_copy(q_hbm.at[0, l, b, :],
                             qv.at[pl.ds(l * NHID, NHID)], sem_q)
            for l in range(L)]

    def fire(g):
        sg, h = divmod(g, NH)
        buf = kbufs[g % 2]
        sem = sems[g % 2]
        return [pltpu.async_copy(
            keys_hbm.at[sg * SG + s, b, pl.ds(h * DH, DH)],
            buf.at[s], sem) for s in range(SG)]

    cps = {0: fire(0)}

    for cp in qcps:
        cp.wait()

    @plsc.parallel_loop(0, DK // (2 * LANES), unroll=4)
    def q_round_body(i):
        q0, q1 = _round_bf16_pair(qv[pl.ds(i * 2 * LANES, LANES)],
                                  qv[pl.ds(i * 2 * LANES + LANES, LANES)])
        qv[pl.ds(i * 2 * LANES, LANES)] = q0
        qv[pl.ds(i * 2 * LANES + LANES, LANES)] = q1

    s0 = jnp.zeros((LANES,), jnp.float32)
    s1 = jnp.zeros((LANES,), jnp.float32)
    accs = None
    for g in range(NH * NGROUPS):
        sg, h = divmod(g, NH)
        if g + 1 < NH * NGROUPS:
            cps[g + 1] = fire(g + 1)
        for cp in cps.pop(g):
            cp.wait()
        buf = kbufs[g % 2]
        if h == 0:
            accs = (jnp.zeros((LANES,), jnp.float32),) * SG
        qoff = h * DH

        def dot_body(i, accs):
            q0 = qv[pl.ds(qoff + i * 2 * LANES, LANES)]
            q1 = qv[pl.ds(qoff + i * 2 * LANES + LANES, LANES)]
            out = []
            for s in range(SG):
                k0, k1 = _round_bf16_pair(
                    buf[s, pl.ds(i * 2 * LANES, LANES)],
                    buf[s, pl.ds(i * 2 * LANES + LANES, LANES)])
                out.append(accs[s] + k0 * q0 + k1 * q1)
            return tuple(out)

        accs = plsc.parallel_loop(0, DH // (2 * LANES), unroll=4,
                                  carry=accs)(dot_body)
        if h == NH - 1:
            for s in range(SG):
                n = sg * SG + s
                score = jnp.sum(accs[s]) * SCALE
                if n < LANES:
                    s0 = jnp.where(idx0 == n, score, s0)
                else:
                    s1 = jnp.where(idx0 == (n - LANES), score, s1)

    m = jnp.maximum(jnp.max(s0), jnp.max(s1))
    e0 = jnp.exp(s0 - m)
    e1 = jnp.exp(s1 - m)
    denom = jnp.sum(e0) + jnp.sum(e1)
    a0 = e0 / denom
    a1 = e1 / denom

    sv[pl.ds(0, LANES)] = a0
    sv[pl.ds(LANES, LANES)] = a1
    pltpu.sync_copy(sv, attn_hbm.at[pl.ds(wid * N, N)])

    r0 = jnp.zeros((LANES,), jnp.int32)
    r1 = jnp.zeros((LANES,), jnp.int32)
    for mi in range(N):
        am_s = a0[mi] if mi < LANES else a1[mi - LANES]
        am = jnp.broadcast_to(am_s, (LANES,))
        r0 = r0 + (am > a0).astype(jnp.int32)
        r1 = r1 + (am > a1).astype(jnp.int32)
        r0 = r0 + ((am == a0) & (idx0 > mi)).astype(jnp.int32)
        r1 = r1 + ((am == a1) & (idx1 > mi)).astype(jnp.int32)

    plsc.store_scatter(tv, [r0], idx0, mask=r0 < TOPK)
    plsc.store_scatter(tv, [r1], idx1, mask=r1 < TOPK)
    pltpu.sync_copy(tv.at[pl.ds(0, TOPK)], topk_hbm.at[pl.ds(wid * TOPK, TOPK)])


@functools.partial(
    pl.kernel,
    mesh=plsc.VectorSubcoreMesh(core_axis_name="c", subcore_axis_name="s"),
    out_type=[
        jax.ShapeDtypeStruct((B_SC * N,), jnp.float32),
        jax.ShapeDtypeStruct((B_SC * TOPK,), jnp.int32),
    ],
    scratch_types=[
        pltpu.VMEM((DK,), jnp.float32),
        pltpu.VMEM((SG, DH), jnp.float32),
        pltpu.VMEM((SG, DH), jnp.float32),
        pltpu.VMEM((N,), jnp.float32),
        pltpu.VMEM((LANES,), jnp.int32),
        pltpu.SemaphoreType.DMA,
        pltpu.SemaphoreType.DMA,
        pltpu.SemaphoreType.DMA,
    ],
    compiler_params=pltpu.CompilerParams(needs_layout_passes=False),
)
def _sc_half(q_hbm, keys_hbm, attn_hbm, topk_hbm, qv, kb0, kb1, sv, tv,
             sem_q, sem0, sem1):
    _sc_body(q_hbm, keys_hbm, attn_hbm, topk_hbm, qv, kb0, kb1, sv, tv,
             sem_q, sem0, sem1)


def _tc_body(keys_ref, q_ref, attn_ref, topk_ref, scores_ref):
    n = pl.program_id(0)
    kr = keys_ref[0].astype(jnp.bfloat16).astype(jnp.float32)
    qr = q_ref[...].astype(jnp.float32)
    partial = jnp.sum(kr * qr, axis=1) * SCALE
    lane = jax.lax.broadcasted_iota(jnp.int32, (B_TC, N), 1)
    scores_ref[...] = jnp.where(lane == n, partial[:, None], scores_ref[...])

    @pl.when(n == N - 1)
    def _():
        s = scores_ref[...]
        m = jnp.max(s, axis=1, keepdims=True)
        e = jnp.exp(s - m)
        att = e / jnp.sum(e, axis=1, keepdims=True)
        attn_ref[...] = att[:, None, :]

        rank = jnp.zeros((B_TC, N), jnp.int32)
        for mi in range(N):
            am = att[:, mi:mi + 1]
            rank = rank + (am > att).astype(jnp.int32)
            rank = rank + ((am == att) & (lane > mi)).astype(jnp.int32)
        cols = []
        for k in range(TOPK):
            cols.append(jnp.sum(jnp.where(rank == k, lane, 0), axis=1,
                                keepdims=True))
        topk_ref[...] = jnp.concatenate(cols, axis=1)


_tc_call = pl.pallas_call(
    _tc_body,
    grid=(N,),
    in_specs=[
        pl.BlockSpec((1, B_TC, DK), lambda n: (n, 0, 0)),
        pl.BlockSpec((B_TC, DK), lambda n: (0, 0)),
    ],
    out_specs=[
        pl.BlockSpec((B_TC, 1, N), lambda n: (0, 0, 0)),
        pl.BlockSpec((B_TC, TOPK), lambda n: (0, 0)),
    ],
    out_shape=[
        jax.ShapeDtypeStruct((B_TC, 1, N), jnp.float32),
        jax.ShapeDtypeStruct((B_TC, TOPK), jnp.int32),
    ],
    scratch_shapes=[pltpu.VMEM((B_TC, N), jnp.float32)],
)


def kernel(query, keys, values):
    del values
    attn_sc_flat, topk_sc_flat = _sc_half(query, keys)
    qf = jnp.transpose(query[:, :, :B_TC, :], (0, 2, 1, 3)).reshape(B_TC, DK)
    attn_tc, topk_tc = _tc_call(keys, qf.astype(jnp.bfloat16))
    attention = jnp.concatenate(
        [attn_tc, attn_sc_flat.reshape(B_SC, 1, N)], axis=0)
    topk_indices = jnp.concatenate(
        [topk_tc, topk_sc_flat.reshape(B_SC, TOPK)], axis=0).T
    return attention, topk_indices
